# Initial kernel scaffold; baseline (speedup 1.0000x reference)
import jax
import jax.numpy as jnp
from jax.experimental import pallas as pl


def kernel(points, W1, b1, W2, b2, gamma, beta):
    raise NotImplementedError("write your pallas kernel here")


# speedup vs baseline: 1.3089x; 1.3089x over previous
import functools
import math

import jax
import jax.numpy as jnp
from jax.experimental import pallas as pl
from jax.experimental.pallas import tpu as pltpu

_EMBED = 384
_GROUPS = 128
_KNN = 64
_HID = 128


def _encoder_body(f0_ref, pts_ref, w1_ref, b1_ref, w2_ref, b2_ref,
                  gamma_ref, beta_ref, cent_ref, tok_ref, dist_ref, x_ref):
    b = pl.program_id(0)
    n = pts_ref.shape[2]
    P = pts_ref[0]
    px, py, pz = P[0:1, :], P[1:2, :], P[2:3, :]
    iota1 = jax.lax.broadcasted_iota(jnp.int32, (1, n), 1)

    def fps_body(g, carry):
        dist, far = carry
        onehot = iota1 == far
        cx = jnp.sum(jnp.where(onehot, px, 0.0))
        cy = jnp.sum(jnp.where(onehot, py, 0.0))
        cz = jnp.sum(jnp.where(onehot, pz, 0.0))
        crow = jnp.concatenate(
            [jnp.full((1, 1), cx), jnp.full((1, 1), cy), jnp.full((1, 1), cz)],
            axis=1)
        cent_ref[0, pl.ds(g, 1), :] = crow
        d = (px - cx) ** 2 + (py - cy) ** 2 + (pz - cz) ** 2
        dist = jnp.minimum(dist, d)
        m = jnp.max(dist)
        far = jnp.min(jnp.where(dist == m, iota1, n)).astype(jnp.int32)
        return dist, far

    dist0 = jnp.full((1, n), jnp.inf, dtype=jnp.float32)
    jax.lax.fori_loop(0, _GROUPS, fps_body, (dist0, f0_ref[b]))

    C = cent_ref[0]
    c2 = jnp.sum(C * C, axis=1, keepdims=True)
    p2 = jnp.sum(P * P, axis=0, keepdims=True)
    G = jax.lax.dot_general(C.astype(jnp.bfloat16), P.astype(jnp.bfloat16),
                            (((1,), (0,)), ((), ())),
                            preferred_element_type=jnp.float32)
    dist_ref[...] = jnp.sqrt(jnp.maximum(c2 + p2 - 2.0 * G, 1e-12))

    iota2 = jax.lax.broadcasted_iota(jnp.int32, (_GROUPS, n), 1)
    cx_col = C[:, 0:1]
    cy_col = C[:, 1:2]
    cz_col = C[:, 2:3]

    def sel_body(k, _):
        dmat = dist_ref[...]
        m = jnp.min(dmat, axis=1, keepdims=True)
        sel = jnp.min(jnp.where(dmat == m, iota2, n), axis=1, keepdims=True)
        onehot = iota2 == sel
        gx = jnp.sum(jnp.where(onehot, px, 0.0), axis=1, keepdims=True)
        gy = jnp.sum(jnp.where(onehot, py, 0.0), axis=1, keepdims=True)
        gz = jnp.sum(jnp.where(onehot, pz, 0.0), axis=1, keepdims=True)
        loc = jnp.concatenate([gx - cx_col, gy - cy_col, gz - cz_col], axis=1)
        x_ref[pl.ds(k, 1), :, :] = loc.reshape(1, _GROUPS, 3)
        dist_ref[...] = jnp.where(onehot, jnp.inf, dmat)
        return 0

    jax.lax.fori_loop(0, _KNN, sel_body, 0)

    X = x_ref[...].reshape(_KNN * _GROUPS, 3)
    A = jax.lax.dot_general(X.astype(jnp.bfloat16),
                            w1_ref[...].astype(jnp.bfloat16),
                            (((1,), (0,)), ((), ())),
                            preferred_element_type=jnp.float32) + b1_ref[...]
    Ag = 0.5 * A * (1.0 + jax.lax.erf(A * (1.0 / math.sqrt(2.0))))
    H = jax.lax.dot_general(Ag.astype(jnp.bfloat16),
                            w2_ref[...].astype(jnp.bfloat16),
                            (((1,), (0,)), ((), ())),
                            preferred_element_type=jnp.float32) + b2_ref[...]
    T = jnp.mean(H.reshape(_KNN, _GROUPS, _EMBED), axis=0)
    mu = jnp.mean(T, axis=1, keepdims=True)
    var = jnp.mean((T - mu) ** 2, axis=1, keepdims=True)
    Tn = (T - mu) / jnp.sqrt(var + 1e-5)
    tok_ref[0, :, :] = Tn * gamma_ref[...] + beta_ref[...]


def _encode(points_t, f0, W1, b1, W2, b2, gamma, beta, interpret=False):
    B, _, N = points_t.shape
    grid_spec = pltpu.PrefetchScalarGridSpec(
        num_scalar_prefetch=1,
        grid=(B,),
        in_specs=[
            pl.BlockSpec((1, 3, N), lambda b, f0: (b, 0, 0)),
            pl.BlockSpec((3, _HID), lambda b, f0: (0, 0)),
            pl.BlockSpec((1, _HID), lambda b, f0: (0, 0)),
            pl.BlockSpec((_HID, _EMBED), lambda b, f0: (0, 0)),
            pl.BlockSpec((1, _EMBED), lambda b, f0: (0, 0)),
            pl.BlockSpec((1, _EMBED), lambda b, f0: (0, 0)),
            pl.BlockSpec((1, _EMBED), lambda b, f0: (0, 0)),
        ],
        out_specs=[
            pl.BlockSpec((1, _GROUPS, 3), lambda b, f0: (b, 0, 0)),
            pl.BlockSpec((1, _GROUPS, _EMBED), lambda b, f0: (b, 0, 0)),
        ],
        scratch_shapes=[
            pltpu.VMEM((_GROUPS, N), jnp.float32),
            pltpu.VMEM((_KNN, _GROUPS, 3), jnp.float32),
        ],
    )
    return pl.pallas_call(
        _encoder_body,
        grid_spec=grid_spec,
        out_shape=[
            jax.ShapeDtypeStruct((B, _GROUPS, 3), jnp.float32),
            jax.ShapeDtypeStruct((B, _GROUPS, _EMBED), jnp.float32),
        ],
        interpret=interpret,
    )(f0, points_t, W1, b1.reshape(1, _HID), W2, b2.reshape(1, _EMBED),
      gamma.reshape(1, _EMBED), beta.reshape(1, _EMBED))


@functools.partial(jax.jit, static_argnames=("interpret",))
def _kernel_impl(points, W1, b1, W2, b2, gamma, beta, interpret=False):
    B, N, _ = points.shape
    f0 = jax.random.randint(jax.random.key(42), (B,), 0, N).astype(jnp.int32)
    points_t = jnp.transpose(points, (0, 2, 1))
    centroids, tokens = _encode(points_t, f0, W1, b1, W2, b2, gamma, beta,
                                interpret=interpret)
    return centroids, tokens


def kernel(points, W1, b1, W2, b2, gamma, beta):
    return _kernel_impl(points, W1, b1, W2, b2, gamma, beta)


# speedup vs baseline: 1.8661x; 1.4257x over previous
import functools
import math

import jax
import jax.numpy as jnp
from jax import lax
from jax.experimental import pallas as pl
from jax.experimental.pallas import tpu as pltpu
from jax.experimental.pallas import tpu_sc as plsc

_EMBED = 384
_GROUPS = 128
_KNN = 64
_HID = 128
_PAD = 128


def _select_body(f0_ref, pts_ref, cent_ref, idx_ref, dist_ref):
    b = pl.program_id(0)
    n = pts_ref.shape[2]
    P = pts_ref[0]
    px, py, pz = P[0:1, :], P[1:2, :], P[2:3, :]
    iota1 = lax.broadcasted_iota(jnp.int32, (1, n), 1)

    def fps_body(g, carry):
        dist, far = carry
        onehot = iota1 == far
        cx = jnp.sum(jnp.where(onehot, px, 0.0))
        cy = jnp.sum(jnp.where(onehot, py, 0.0))
        cz = jnp.sum(jnp.where(onehot, pz, 0.0))
        crow = jnp.concatenate(
            [jnp.full((1, 1), cx), jnp.full((1, 1), cy), jnp.full((1, 1), cz)],
            axis=1)
        cent_ref[0, pl.ds(g, 1), :] = crow
        d = (px - cx) ** 2 + (py - cy) ** 2 + (pz - cz) ** 2
        dist = jnp.minimum(dist, d)
        m = jnp.max(dist)
        far = jnp.min(jnp.where(dist == m, iota1, n)).astype(jnp.int32)
        return dist, far

    dist0 = jnp.full((1, n), jnp.inf, dtype=jnp.float32)
    lax.fori_loop(0, _GROUPS, fps_body, (dist0, f0_ref[b]))

    C = cent_ref[0]
    c2 = jnp.sum(C * C, axis=1, keepdims=True)
    p2 = jnp.sum(P * P, axis=0, keepdims=True)
    G = lax.dot_general(C.astype(jnp.bfloat16), P.astype(jnp.bfloat16),
                        (((1,), (0,)), ((), ())),
                        preferred_element_type=jnp.float32)
    dist_ref[...] = jnp.sqrt(jnp.maximum(c2 + p2 - 2.0 * G, 1e-12))

    iota2 = lax.broadcasted_iota(jnp.int32, (_GROUPS, n), 1)
    iota_k = lax.broadcasted_iota(jnp.int32, (_GROUPS, _KNN), 1)
    base = b * n

    def sel_body(k, acc):
        dmat = dist_ref[...]
        m = jnp.min(dmat, axis=1, keepdims=True)
        sel = jnp.min(jnp.where(dmat == m, iota2, n), axis=1, keepdims=True)
        acc = jnp.where(iota_k == k, sel + base, acc)
        dist_ref[...] = jnp.where(iota2 == sel, jnp.inf, dmat)
        return acc

    acc0 = jnp.zeros((_GROUPS, _KNN), jnp.int32)
    idx_ref[0] = lax.fori_loop(0, _KNN, sel_body, acc0)


def _select(points_t, f0, interpret=False):
    B, _, N = points_t.shape
    grid_spec = pltpu.PrefetchScalarGridSpec(
        num_scalar_prefetch=1,
        grid=(B,),
        in_specs=[pl.BlockSpec((1, 3, N), lambda b, f0: (b, 0, 0))],
        out_specs=[
            pl.BlockSpec((1, _GROUPS, 3), lambda b, f0: (b, 0, 0)),
            pl.BlockSpec((1, _GROUPS, _KNN), lambda b, f0: (b, 0, 0)),
        ],
        scratch_shapes=[pltpu.VMEM((_GROUPS, N), jnp.float32)],
    )
    return pl.pallas_call(
        _select_body,
        grid_spec=grid_spec,
        out_shape=[
            jax.ShapeDtypeStruct((B, _GROUPS, 3), jnp.float32),
            jax.ShapeDtypeStruct((B, _GROUPS, _KNN), jnp.int32),
        ],
        interpret=interpret,
    )(f0, points_t)


def _sc_gather(table, idx):
    M = idx.shape[0]
    info = plsc.get_sparse_core_info()
    nw = info.num_cores * info.num_subcores
    m_per_w = M // nw
    chunk = 512
    n_chunks = m_per_w // chunk
    mesh = plsc.VectorSubcoreMesh(core_axis_name="c", subcore_axis_name="s")

    @functools.partial(
        pl.kernel, mesh=mesh,
        out_type=jax.ShapeDtypeStruct((M, _PAD), jnp.float32),
        scratch_types=[
            pltpu.VMEM((chunk,), jnp.int32),
            pltpu.VMEM((chunk, _PAD), jnp.float32),
            pltpu.SemaphoreType.DMA,
        ],
    )
    def k(table_hbm, idx_hbm, out_hbm, idx_v, rows_v, sem):
        wid = lax.axis_index("s") * info.num_cores + lax.axis_index("c")
        base = wid * m_per_w
        for c in range(n_chunks):
            cbase = base + c * chunk
            pltpu.sync_copy(idx_hbm.at[pl.ds(cbase, chunk)], idx_v)
            pltpu.async_copy(table_hbm.at[idx_v], rows_v, sem).wait()
            pltpu.sync_copy(rows_v, out_hbm.at[pl.ds(cbase, chunk)])

    return k(table, idx)


def _mlp_body(gath_ref, cent_ref, w1_ref, b1_ref, w2_ref, b2_ref,
              gamma_ref, beta_ref, tok_ref):
    Gt = gath_ref[0].reshape(_GROUPS, _KNN, _PAD)
    C = cent_ref[0].reshape(_GROUPS, 1, _PAD)
    X = (Gt - C).reshape(_GROUPS * _KNN, _PAD)
    A = lax.dot_general(X.astype(jnp.bfloat16),
                        w1_ref[...].astype(jnp.bfloat16),
                        (((1,), (0,)), ((), ())),
                        preferred_element_type=jnp.float32) + b1_ref[...]
    Ag = 0.5 * A * (1.0 + lax.erf(A * (1.0 / math.sqrt(2.0))))
    H = lax.dot_general(Ag.astype(jnp.bfloat16),
                        w2_ref[...].astype(jnp.bfloat16),
                        (((1,), (0,)), ((), ())),
                        preferred_element_type=jnp.float32) + b2_ref[...]
    T = jnp.mean(H.reshape(_GROUPS, _KNN, _EMBED), axis=1)
    mu = jnp.mean(T, axis=1, keepdims=True)
    var = jnp.mean((T - mu) ** 2, axis=1, keepdims=True)
    Tn = (T - mu) / jnp.sqrt(var + 1e-5)
    tok_ref[0, :, :] = Tn * gamma_ref[...] + beta_ref[...]


def _mlp(gathered, cent_pad, W1p, b1, W2, b2, gamma, beta, B, interpret=False):
    return pl.pallas_call(
        _mlp_body,
        grid=(B,),
        in_specs=[
            pl.BlockSpec((1, _GROUPS * _KNN, _PAD), lambda b: (b, 0, 0)),
            pl.BlockSpec((1, _GROUPS, _PAD), lambda b: (b, 0, 0)),
            pl.BlockSpec((_PAD, _HID), lambda b: (0, 0)),
            pl.BlockSpec((1, _HID), lambda b: (0, 0)),
            pl.BlockSpec((_HID, _EMBED), lambda b: (0, 0)),
            pl.BlockSpec((1, _EMBED), lambda b: (0, 0)),
            pl.BlockSpec((1, _EMBED), lambda b: (0, 0)),
            pl.BlockSpec((1, _EMBED), lambda b: (0, 0)),
        ],
        out_specs=pl.BlockSpec((1, _GROUPS, _EMBED), lambda b: (b, 0, 0)),
        out_shape=jax.ShapeDtypeStruct((B, _GROUPS, _EMBED), jnp.float32),
        interpret=interpret,
    )(gathered.reshape(B, _GROUPS * _KNN, _PAD), cent_pad, W1p,
      b1.reshape(1, _HID), W2, b2.reshape(1, _EMBED),
      gamma.reshape(1, _EMBED), beta.reshape(1, _EMBED))


@functools.partial(jax.jit, static_argnames=("interpret", "sc"))
def _kernel_impl(points, W1, b1, W2, b2, gamma, beta,
                 interpret=False, sc=True):
    B, N, _ = points.shape
    f0 = jax.random.randint(jax.random.key(42), (B,), 0, N).astype(jnp.int32)
    points_t = jnp.transpose(points, (0, 2, 1))
    centroids, idx = _select(points_t, f0, interpret=interpret)

    table = jnp.pad(points, ((0, 0), (0, 0), (0, _PAD - 3))).reshape(B * N, _PAD)
    flat_idx = idx.reshape(B * _GROUPS * _KNN)
    if sc:
        gathered = _sc_gather(table, flat_idx)
    else:
        gathered = table[flat_idx]
    gathered = gathered.reshape(B, _GROUPS * _KNN, _PAD)

    cent_pad = jnp.pad(centroids, ((0, 0), (0, 0), (0, _PAD - 3)))
    W1p = jnp.pad(W1, ((0, _PAD - 3), (0, 0)))
    tokens = _mlp(gathered, cent_pad, W1p, b1, W2, b2, gamma, beta, B,
                  interpret=interpret)
    return centroids, tokens


def kernel(points, W1, b1, W2, b2, gamma, beta):
    return _kernel_impl(points, W1, b1, W2, b2, gamma, beta)
